# baseline (speedup 1.0000x reference)
import functools

import jax
import jax.numpy as jnp
from jax.experimental import pallas as pl
from jax.experimental.pallas import tpu as pltpu

TOKENS = 32768
HIDDEN = 1024
NUM_EXPERTS = 8
TOP_K = 2
AUX_LOSS_COEF = 0.01

BLOCK_M = 2048


def _router_kernel(x_ref, w_ref, b_ref, idx_ref, scores_ref, mask_ref,
                   aux_ref, fsum_ref, msum_ref):
    step = pl.program_id(0)
    nsteps = pl.num_programs(0)

    logits = jax.lax.dot_general(
        x_ref[...], w_ref[...], (((1,), (0,)), ((), ())),
        preferred_element_type=jnp.float32)
    logits = logits + b_ref[...]

    eids = jax.lax.broadcasted_iota(jnp.int32, logits.shape, 1)
    top1 = jnp.max(logits, axis=1, keepdims=True)
    is1 = logits == top1
    idx1 = jnp.min(jnp.where(is1, eids, NUM_EXPERTS), axis=1, keepdims=True)
    mask1 = (eids == idx1).astype(jnp.float32)
    masked = jnp.where(eids == idx1, -jnp.inf, logits)
    top2 = jnp.max(masked, axis=1, keepdims=True)
    is2 = masked == top2
    idx2 = jnp.min(jnp.where(is2, eids, NUM_EXPERTS), axis=1, keepdims=True)
    mask2 = (eids == idx2).astype(jnp.float32)

    idx_ref[...] = jnp.concatenate([idx1, idx2], axis=1)

    e2 = jnp.exp(top2 - top1)
    s1 = 1.0 / (1.0 + e2)
    scores_ref[...] = jnp.concatenate([s1, 1.0 - s1], axis=1)

    mask_ref[...] = jnp.concatenate([mask1, mask2], axis=1)

    p = jnp.exp(logits - top1)
    p = p / jnp.sum(p, axis=1, keepdims=True)

    f_part = jnp.sum(mask1 + mask2, axis=0, keepdims=True)
    m_part = jnp.sum(p, axis=0, keepdims=True)

    @pl.when(step == 0)
    def _init():
        fsum_ref[...] = jnp.zeros_like(fsum_ref)
        msum_ref[...] = jnp.zeros_like(msum_ref)

    fsum_ref[...] += f_part
    msum_ref[...] += m_part

    @pl.when(step == nsteps - 1)
    def _fin():
        f_i = fsum_ref[...] / (TOKENS * TOP_K)
        m_i = msum_ref[...] / TOKENS
        aux_ref[...] = (AUX_LOSS_COEF / NUM_EXPERTS) * jnp.sum(
            f_i * m_i, keepdims=True).reshape(1, 1)


@functools.partial(jax.jit, static_argnames=())
def kernel(x, W, b):
    grid = TOKENS // BLOCK_M
    idx, scores, mask2d, aux = pl.pallas_call(
        _router_kernel,
        grid=(grid,),
        in_specs=[
            pl.BlockSpec((BLOCK_M, HIDDEN), lambda i: (i, 0)),
            pl.BlockSpec((HIDDEN, NUM_EXPERTS), lambda i: (0, 0)),
            pl.BlockSpec((1, NUM_EXPERTS), lambda i: (0, 0)),
        ],
        out_specs=[
            pl.BlockSpec((BLOCK_M, TOP_K), lambda i: (i, 0)),
            pl.BlockSpec((BLOCK_M, TOP_K), lambda i: (i, 0)),
            pl.BlockSpec((BLOCK_M, TOP_K * NUM_EXPERTS), lambda i: (i, 0)),
            pl.BlockSpec((1, 1), lambda i: (0, 0)),
        ],
        out_shape=[
            jax.ShapeDtypeStruct((TOKENS, TOP_K), jnp.int32),
            jax.ShapeDtypeStruct((TOKENS, TOP_K), jnp.float32),
            jax.ShapeDtypeStruct((TOKENS, TOP_K * NUM_EXPERTS), jnp.float32),
            jax.ShapeDtypeStruct((1, 1), jnp.float32),
        ],
        scratch_shapes=[
            pltpu.VMEM((1, NUM_EXPERTS), jnp.float32),
            pltpu.VMEM((1, NUM_EXPERTS), jnp.float32),
        ],
    )(x, W, b.reshape(1, NUM_EXPERTS))
    expert_mask = mask2d.reshape(TOKENS, TOP_K, NUM_EXPERTS)
    return idx, scores, expert_mask, aux[0, 0]


# speedup vs baseline: 2.1331x; 2.1331x over previous
import functools

import jax
import jax.numpy as jnp
from jax.experimental import pallas as pl
from jax.experimental.pallas import tpu as pltpu

TOKENS = 32768
HIDDEN = 1024
NUM_EXPERTS = 8
TOP_K = 2
AUX_LOSS_COEF = 0.01

BLOCK_M = 2048


def _router_kernel(x_ref, w_ref, b_ref, idx_ref, scores_ref, mask_ref,
                   aux_ref, fsum_ref, msum_ref):
    step = pl.program_id(0)
    nsteps = pl.num_programs(0)

    logits = jax.lax.dot_general(
        w_ref[...], x_ref[...], (((0,), (1,)), ((), ())),
        preferred_element_type=jnp.float32)
    logits = logits + b_ref[...]

    eids = jax.lax.broadcasted_iota(jnp.int32, logits.shape, 0)
    top1 = jnp.max(logits, axis=0, keepdims=True)
    idx1 = jnp.min(jnp.where(logits == top1, eids, NUM_EXPERTS),
                   axis=0, keepdims=True)
    hit1 = eids == idx1
    mask1 = hit1.astype(jnp.float32)
    masked = jnp.where(hit1, -jnp.inf, logits)
    top2 = jnp.max(masked, axis=0, keepdims=True)
    idx2 = jnp.min(jnp.where(masked == top2, eids, NUM_EXPERTS),
                   axis=0, keepdims=True)
    mask2 = (eids == idx2).astype(jnp.float32)

    idx_ref[...] = jnp.concatenate([idx1, idx2], axis=0)

    e2 = jnp.exp(top2 - top1)
    s1 = 1.0 / (1.0 + e2)
    scores_ref[...] = jnp.concatenate([s1, 1.0 - s1], axis=0)

    mask_ref[...] = jnp.concatenate([mask1, mask2], axis=0)

    p = jnp.exp(logits - top1)
    p = p / jnp.sum(p, axis=0, keepdims=True)

    f_part = jnp.sum(mask1 + mask2, axis=1, keepdims=True)
    m_part = jnp.sum(p, axis=1, keepdims=True)

    @pl.when(step == 0)
    def _init():
        fsum_ref[...] = jnp.zeros_like(fsum_ref)
        msum_ref[...] = jnp.zeros_like(msum_ref)

    fsum_ref[...] += f_part
    msum_ref[...] += m_part

    @pl.when(step == nsteps - 1)
    def _fin():
        f_i = fsum_ref[...] / (TOKENS * TOP_K)
        m_i = msum_ref[...] / TOKENS
        aux_ref[...] = (AUX_LOSS_COEF / NUM_EXPERTS) * jnp.sum(
            f_i * m_i, keepdims=True).reshape(1, 1)


@functools.partial(jax.jit, static_argnames=())
def kernel(x, W, b):
    grid = TOKENS // BLOCK_M
    idx_t, scores_t, mask_t, aux = pl.pallas_call(
        _router_kernel,
        grid=(grid,),
        in_specs=[
            pl.BlockSpec((BLOCK_M, HIDDEN), lambda i: (i, 0)),
            pl.BlockSpec((HIDDEN, NUM_EXPERTS), lambda i: (0, 0)),
            pl.BlockSpec((NUM_EXPERTS, 1), lambda i: (0, 0)),
        ],
        out_specs=[
            pl.BlockSpec((TOP_K, BLOCK_M), lambda i: (0, i)),
            pl.BlockSpec((TOP_K, BLOCK_M), lambda i: (0, i)),
            pl.BlockSpec((TOP_K * NUM_EXPERTS, BLOCK_M), lambda i: (0, i)),
            pl.BlockSpec((1, 1), lambda i: (0, 0)),
        ],
        out_shape=[
            jax.ShapeDtypeStruct((TOP_K, TOKENS), jnp.int32),
            jax.ShapeDtypeStruct((TOP_K, TOKENS), jnp.float32),
            jax.ShapeDtypeStruct((TOP_K * NUM_EXPERTS, TOKENS), jnp.float32),
            jax.ShapeDtypeStruct((1, 1), jnp.float32),
        ],
        scratch_shapes=[
            pltpu.VMEM((NUM_EXPERTS, 1), jnp.float32),
            pltpu.VMEM((NUM_EXPERTS, 1), jnp.float32),
        ],
    )(x, W, b.reshape(NUM_EXPERTS, 1))
    expert_indices = idx_t.T
    scores = scores_t.T
    expert_mask = mask_t.reshape(TOP_K, NUM_EXPERTS, TOKENS).transpose(2, 0, 1)
    return expert_indices, scores, expert_mask, aux[0, 0]
